# baseline (speedup 1.0000x reference)
import jax
import jax.numpy as jnp
from jax.experimental import pallas as pl
from jax.experimental.pallas import tpu as pltpu

L = 32
S = 64
D_L = 128
D_ATT = 128
NEG = -1e9


def _prelude_kernel(g_ref, wq_ref, wk_ref, vg_ref, x_ref, wk1_ref,
                    gw_ref, p1_ref):
    g = g_ref[...]
    q = jnp.dot(g, wq_ref[...], preferred_element_type=jnp.float32)
    k = jnp.dot(g, wk_ref[...], preferred_element_type=jnp.float32)
    t = jnp.tanh(q[:, None, :] + k[None, :, :])
    s = jnp.dot(t.reshape(L * L, D_ATT), vg_ref[...],
                preferred_element_type=jnp.float32).reshape(L, L)
    s = s - jnp.max(s, axis=-1, keepdims=True)
    e = jnp.exp(s)
    gw_ref[...] = e / jnp.sum(e, axis=-1, keepdims=True)
    x2d = x_ref[...].reshape(L * S, D_L)
    p1_ref[...] = jnp.tanh(
        jnp.dot(x2d, wk1_ref[...], preferred_element_type=jnp.float32)
    ).reshape(L, S, D_ATT)


def _layer1_kernel(x_ref, p1_ref, wk2_ref, wsel_ref, wroot_ref,
                   gw_ref, spk_ref, len_ref,
                   lw_ref, x1_ref):
    j = pl.program_id(0)
    p2j = jnp.tanh(jnp.dot(x_ref[j], wk2_ref[...],
                           preferred_element_type=jnp.float32))
    len_j = len_ref[j]
    sp_j = spk_ref[j]
    t_idx = jax.lax.broadcasted_iota(jnp.int32, (S, S), 1)
    s_idx = jax.lax.broadcasted_iota(jnp.int32, (S, 1), 0)
    tmask = t_idx < len_j
    scale = 1.0 / jnp.sqrt(jnp.float32(D_ATT))

    def body(i, accs):
        a00, a01, a10, a11 = accs
        p1i = p1_ref[i]
        sc = jax.lax.dot_general(
            p1i, p2j, (((1,), (1,)), ((), ())),
            preferred_element_type=jnp.float32) * scale
        sc = jnp.where(tmask, sc, NEG)
        sc = sc - jnp.max(sc, axis=-1, keepdims=True)
        e = jnp.exp(sc)
        lwi = e / jnp.sum(e, axis=-1, keepdims=True)
        lwi = lwi * (s_idx < len_ref[i]).astype(jnp.float32)
        lw_ref[i, 0] = lwi
        msg = jnp.dot(lwi, x_ref[i],
                      preferred_element_type=jnp.float32)
        msg = msg * gw_ref[i, j]
        sp_i = spk_ref[i]
        fwd = (i < j).astype(jnp.float32)
        bwd = 1.0 - fwd
        s0 = (sp_i == 0).astype(jnp.float32)
        s1 = 1.0 - s0
        return (a00 + msg * (s0 * fwd), a01 + msg * (s0 * bwd),
                a10 + msg * (s1 * fwd), a11 + msg * (s1 * bwd))

    zero = jnp.zeros((S, D_L), jnp.float32)
    a00, a01, a10, a11 = jax.lax.fori_loop(
        0, L, body, (zero, zero, zero, zero))

    is0 = sp_j == 0

    def rel(a, d):
        return jnp.where(is0, wsel_ref[a, 0, d], wsel_ref[a, 1, d])

    agg = (jnp.dot(a00, rel(0, 0), preferred_element_type=jnp.float32)
           + jnp.dot(a01, rel(0, 1), preferred_element_type=jnp.float32)
           + jnp.dot(a10, rel(1, 0), preferred_element_type=jnp.float32)
           + jnp.dot(a11, rel(1, 1), preferred_element_type=jnp.float32))
    x1_ref[0] = jnp.dot(x_ref[j], wroot_ref[...],
                        preferred_element_type=jnp.float32) + agg


def _layer2_kernel(lw_ref, x1_ref, wrel_ref, wroot_ref, x2_ref):
    j = pl.program_id(0)

    def body(i, acc):
        return acc + jnp.dot(lw_ref[i, 0], x1_ref[i],
                             preferred_element_type=jnp.float32)

    acc = jax.lax.fori_loop(0, L, body, jnp.zeros((S, D_L), jnp.float32))
    x2_ref[0] = (jnp.dot(x1_ref[j], wroot_ref[...],
                         preferred_element_type=jnp.float32)
                 + jnp.dot(acc, wrel_ref[...],
                           preferred_element_type=jnp.float32))


@jax.jit
def kernel(global_features, local_features, speaker, length, Wq_g, Wk_g,
           v_g, Wk1_l, Wk2_l, W_rel1, W_root1, W_rel2, W_root2):
    f32 = jnp.float32
    x = local_features.astype(f32)
    wsel = jnp.concatenate([W_rel1[0:4], W_rel1[64:68]], axis=0)
    wsel = wsel.reshape(2, 2, 2, D_L, D_L)
    vg2 = v_g.reshape(D_ATT, 1)
    spk = speaker.astype(jnp.int32)
    lng = length.astype(jnp.int32)
    wrel2 = W_rel2.reshape(D_L, D_L)

    gw, p1 = pl.pallas_call(
        _prelude_kernel,
        out_shape=(jax.ShapeDtypeStruct((L, L), f32),
                   jax.ShapeDtypeStruct((L, S, D_ATT), f32)),
    )(global_features, Wq_g, Wk_g, vg2, x, Wk1_l)

    def full(arr):
        n = arr.ndim
        return pl.BlockSpec(arr.shape, lambda j, n=n: (0,) * n)

    smem = pl.BlockSpec(memory_space=pltpu.SMEM)

    lw, x1 = pl.pallas_call(
        _layer1_kernel,
        grid=(L,),
        in_specs=[full(x), full(p1), full(Wk2_l), full(wsel), full(W_root1),
                  smem, smem, smem],
        out_specs=(pl.BlockSpec((L, 1, S, S), lambda j: (0, j, 0, 0)),
                   pl.BlockSpec((1, S, D_L), lambda j: (j, 0, 0))),
        out_shape=(jax.ShapeDtypeStruct((L, L, S, S), f32),
                   jax.ShapeDtypeStruct((L, S, D_L), f32)),
    )(x, p1, Wk2_l, wsel, W_root1, gw, spk, lng)

    x2 = pl.pallas_call(
        _layer2_kernel,
        grid=(L,),
        in_specs=[pl.BlockSpec((L, 1, S, S), lambda j: (0, j, 0, 0)),
                  full(x1), full(wrel2), full(W_root2)],
        out_specs=pl.BlockSpec((1, S, D_L), lambda j: (j, 0, 0)),
        out_shape=jax.ShapeDtypeStruct((L, S, D_L), f32),
    )(lw, x1, wrel2, W_root2)

    return x2


# speedup vs baseline: 3.6816x; 3.6816x over previous
import jax
import jax.numpy as jnp
from jax.experimental import pallas as pl
from jax.experimental.pallas import tpu as pltpu

L = 32
S = 64
D_L = 128
D_ATT = 128
NEG = -1e9


def _prelude_kernel(g_ref, wq_ref, wk_ref, vg_ref, x_ref, wk1_ref, lng_ref,
                    gwt_ref, p1_ref, qmask_ref):
    g = g_ref[...]
    q = jnp.dot(g, wq_ref[...], preferred_element_type=jnp.float32)
    k = jnp.dot(g, wk_ref[...], preferred_element_type=jnp.float32)
    t = jnp.tanh(q[:, None, :] + k[None, :, :])
    s = jnp.dot(t.reshape(L * L, D_ATT), vg_ref[...],
                preferred_element_type=jnp.float32).reshape(L, L)
    s = s - jnp.max(s, axis=-1, keepdims=True)
    e = jnp.exp(s)
    gw = e / jnp.sum(e, axis=-1, keepdims=True)
    gwt_ref[...] = gw.T[:, :, None]
    x2d = x_ref[...].reshape(L * S, D_L)
    p1_ref[...] = jnp.tanh(
        jnp.dot(x2d, wk1_ref[...], preferred_element_type=jnp.float32)
    ).reshape(L, S, D_ATT)
    s_iota = jax.lax.broadcasted_iota(jnp.int32, (L, S), 1)
    qmask_ref[...] = (s_iota < lng_ref[...]).astype(jnp.float32)[None]


def _layer1_kernel(x_ref, p1_ref, wk2_ref, wsel_ref, wroot_ref, gwt_ref,
                   qmask_ref, spkf_ref, spk_ref, len_ref,
                   lw_ref, x1_ref):
    j = pl.program_id(0)
    p2j = jnp.tanh(jnp.dot(x_ref[j], wk2_ref[...],
                           preferred_element_type=jnp.float32))
    sc3 = jax.lax.dot_general(
        p2j, p1_ref[...], (((1,), (2,)), ((), ())),
        preferred_element_type=jnp.float32) * (1.0 / jnp.sqrt(
            jnp.float32(D_ATT)))
    t_iota = jax.lax.broadcasted_iota(jnp.int32, (S, L, S), 0)
    sc3 = jnp.where(t_iota < len_ref[j], sc3, NEG)
    sc3 = sc3 - jnp.max(sc3, axis=0, keepdims=True)
    e = jnp.exp(sc3)
    lw3 = e / jnp.sum(e, axis=0, keepdims=True)
    lw3 = lw3 * qmask_ref[...]
    lw_ref[0] = lw3
    ew3 = lw3 * gwt_ref[...]
    msg = jax.lax.dot_general(
        ew3, x_ref[...], (((0,), (1,)), ((1,), (0,))),
        preferred_element_type=jnp.float32)
    spr = spkf_ref[...]
    ilt = (jax.lax.broadcasted_iota(jnp.int32, (1, L), 1) < j
           ).astype(jnp.float32)
    wg = jnp.concatenate([(1.0 - spr) * ilt, (1.0 - spr) * (1.0 - ilt),
                          spr * ilt, spr * (1.0 - ilt)], axis=0)
    acc4 = jax.lax.dot_general(
        wg, msg, (((1,), (0,)), ((), ())),
        preferred_element_type=jnp.float32)
    is0 = spk_ref[j] == 0

    def rel(a, d):
        return jnp.where(is0, wsel_ref[a, 0, d], wsel_ref[a, 1, d])

    agg = (jnp.dot(acc4[0], rel(0, 0), preferred_element_type=jnp.float32)
           + jnp.dot(acc4[1], rel(0, 1), preferred_element_type=jnp.float32)
           + jnp.dot(acc4[2], rel(1, 0), preferred_element_type=jnp.float32)
           + jnp.dot(acc4[3], rel(1, 1), preferred_element_type=jnp.float32))
    x1_ref[0] = jnp.dot(x_ref[j], wroot_ref[...],
                        preferred_element_type=jnp.float32) + agg


def _layer2_kernel(lw_ref, x1_ref, wrel_ref, wroot_ref, x2_ref):
    j = pl.program_id(0)
    msg = jax.lax.dot_general(
        lw_ref[0], x1_ref[...], (((0,), (1,)), ((1,), (0,))),
        preferred_element_type=jnp.float32)
    msum = jnp.sum(msg, axis=0)
    x2_ref[0] = (jnp.dot(x1_ref[j], wroot_ref[...],
                         preferred_element_type=jnp.float32)
                 + jnp.dot(msum, wrel_ref[...],
                           preferred_element_type=jnp.float32))


@jax.jit
def kernel(global_features, local_features, speaker, length, Wq_g, Wk_g,
           v_g, Wk1_l, Wk2_l, W_rel1, W_root1, W_rel2, W_root2):
    f32 = jnp.float32
    x = local_features.astype(f32)
    wsel = jnp.concatenate([W_rel1[0:4], W_rel1[64:68]], axis=0)
    wsel = wsel.reshape(2, 2, 2, D_L, D_L)
    vg2 = v_g.reshape(D_ATT, 1)
    spk = speaker.astype(jnp.int32)
    lng = length.astype(jnp.int32)
    spkf = speaker.astype(f32).reshape(1, L)
    lng2 = lng.reshape(L, 1)
    wrel2 = W_rel2.reshape(D_L, D_L)

    gwt, p1, qmask = pl.pallas_call(
        _prelude_kernel,
        out_shape=(jax.ShapeDtypeStruct((L, L, 1), f32),
                   jax.ShapeDtypeStruct((L, S, D_ATT), f32),
                   jax.ShapeDtypeStruct((1, L, S), f32)),
    )(global_features, Wq_g, Wk_g, vg2, x, Wk1_l, lng2)

    def full(arr):
        n = arr.ndim
        return pl.BlockSpec(arr.shape, lambda j, n=n: (0,) * n)

    smem = pl.BlockSpec(memory_space=pltpu.SMEM)

    lw, x1 = pl.pallas_call(
        _layer1_kernel,
        grid=(L,),
        in_specs=[full(x), full(p1), full(Wk2_l), full(wsel), full(W_root1),
                  pl.BlockSpec((1, L, 1), lambda j: (j, 0, 0)),
                  full(qmask), full(spkf), smem, smem],
        out_specs=(pl.BlockSpec((1, S, L, S), lambda j: (j, 0, 0, 0)),
                   pl.BlockSpec((1, S, D_L), lambda j: (j, 0, 0))),
        out_shape=(jax.ShapeDtypeStruct((L, S, L, S), f32),
                   jax.ShapeDtypeStruct((L, S, D_L), f32)),
    )(x, p1, Wk2_l, wsel, W_root1, gwt, qmask, spkf, spk, lng)

    x2 = pl.pallas_call(
        _layer2_kernel,
        grid=(L,),
        in_specs=[pl.BlockSpec((1, S, L, S), lambda j: (j, 0, 0, 0)),
                  full(x1), full(wrel2), full(W_root2)],
        out_specs=pl.BlockSpec((1, S, D_L), lambda j: (j, 0, 0)),
        out_shape=jax.ShapeDtypeStruct((L, S, D_L), f32),
    )(lw, x1, wrel2, W_root2)

    return x2


# speedup vs baseline: 7.4282x; 2.0177x over previous
import jax
import jax.numpy as jnp
from jax.experimental import pallas as pl
from jax.experimental.pallas import tpu as pltpu

L = 32
S = 64
D_L = 128
D_ATT = 128
LS = L * S


def _prelude_kernel(g_ref, wq_ref, wk_ref, vg_ref, x_ref, wk1_ref, wk2_ref,
                    gwt_ref, p1_ref, p2t_ref):
    g = g_ref[...]
    q = jnp.dot(g, wq_ref[...], preferred_element_type=jnp.float32)
    k = jnp.dot(g, wk_ref[...], preferred_element_type=jnp.float32)
    t = jnp.tanh(q[:, None, :] + k[None, :, :])
    s = jnp.sum(t * vg_ref[0][None, None, :], axis=-1)
    s = s - jnp.max(s, axis=-1, keepdims=True)
    e = jnp.exp(s)
    gw = e / jnp.sum(e, axis=-1, keepdims=True)
    gwt_ref[...] = gw.T
    x2d = x_ref[...]
    p1_ref[...] = jnp.tanh(
        jnp.dot(x2d, wk1_ref[...], preferred_element_type=jnp.float32))
    p2 = jnp.tanh(
        jnp.dot(x2d, wk2_ref[...], preferred_element_type=jnp.float32))
    p2t_ref[...] = jnp.transpose(p2.reshape(L, S, D_ATT), (0, 2, 1))


def _layer1_kernel(x_ref, p1_ref, p2t_ref, wsel_ref, wroot_ref, gwt_ref,
                   qmask_ref, spkf_ref, spk_ref, len_ref,
                   lw_ref, x1_ref):
    j = pl.program_id(0)
    sc = jnp.dot(p1_ref[...], p2t_ref[0],
                 preferred_element_type=jnp.float32) * (
                     1.0 / jnp.sqrt(jnp.float32(D_ATT)))
    t_iota = jax.lax.broadcasted_iota(jnp.int32, (LS, S), 1)
    e = jnp.exp(sc) * (t_iota < len_ref[j]).astype(jnp.float32)
    ssum = jnp.dot(e, jnp.ones((S, 1), jnp.float32),
                   preferred_element_type=jnp.float32)
    lw = e * (qmask_ref[...] / ssum)
    lw_ref[0] = lw
    msg = jax.lax.dot_general(
        lw.reshape(L, S, S), x_ref[...].reshape(L, S, D_L),
        (((2,), (1,)), ((0,), (0,))),
        preferred_element_type=jnp.float32)
    spr = spkf_ref[...]
    ilt = (jax.lax.broadcasted_iota(jnp.int32, (1, L), 1) < j
           ).astype(jnp.float32)
    wg = jnp.concatenate([(1.0 - spr) * ilt, (1.0 - spr) * (1.0 - ilt),
                          spr * ilt, spr * (1.0 - ilt)],
                         axis=0) * gwt_ref[0]
    acc4 = jax.lax.dot_general(
        wg, msg, (((1,), (0,)), ((), ())),
        preferred_element_type=jnp.float32)
    is0 = spk_ref[j] == 0

    def rel(a, d):
        return jnp.where(is0, wsel_ref[a, 0, d], wsel_ref[a, 1, d])

    agg = (jnp.dot(acc4[0], rel(0, 0), preferred_element_type=jnp.float32)
           + jnp.dot(acc4[1], rel(0, 1), preferred_element_type=jnp.float32)
           + jnp.dot(acc4[2], rel(1, 0), preferred_element_type=jnp.float32)
           + jnp.dot(acc4[3], rel(1, 1), preferred_element_type=jnp.float32))
    x1_ref[0] = jnp.dot(x_ref[pl.ds(j * S, S)], wroot_ref[...],
                        preferred_element_type=jnp.float32) + agg


def _layer2_kernel(lw_ref, x1_ref, wrel_ref, wroot_ref, x2_ref):
    j = pl.program_id(0)
    msg = jax.lax.dot_general(
        lw_ref[0].reshape(L, S, S), x1_ref[...].reshape(L, S, D_L),
        (((2,), (1,)), ((0,), (0,))),
        preferred_element_type=jnp.float32)
    msum = jnp.sum(msg, axis=0)
    x2_ref[0] = (jnp.dot(x1_ref[pl.ds(j * S, S)], wroot_ref[...],
                         preferred_element_type=jnp.float32)
                 + jnp.dot(msum, wrel_ref[...],
                           preferred_element_type=jnp.float32))


@jax.jit
def kernel(global_features, local_features, speaker, length, Wq_g, Wk_g,
           v_g, Wk1_l, Wk2_l, W_rel1, W_root1, W_rel2, W_root2):
    f32 = jnp.float32
    x2d = local_features.astype(f32).reshape(LS, D_L)
    wsel = jnp.concatenate([W_rel1[0:4], W_rel1[64:68]], axis=0)
    wsel = wsel.reshape(2, 2, 2, D_L, D_L)
    vg2 = v_g.reshape(1, D_ATT)
    spk = speaker.astype(jnp.int32)
    lng = length.astype(jnp.int32)
    spkf = speaker.astype(f32).reshape(1, L)
    wrel2 = W_rel2.reshape(D_L, D_L)
    qmask = (jnp.arange(S, dtype=jnp.int32)[None, :] < lng[:, None]
             ).astype(f32).reshape(LS, 1)

    gwt, p1, p2t = pl.pallas_call(
        _prelude_kernel,
        out_shape=(jax.ShapeDtypeStruct((L, L), f32),
                   jax.ShapeDtypeStruct((LS, D_ATT), f32),
                   jax.ShapeDtypeStruct((L, D_ATT, S), f32)),
    )(global_features, Wq_g, Wk_g, vg2, x2d, Wk1_l, Wk2_l)
    gwt3 = gwt.reshape(L, 1, L)

    def full(arr):
        n = arr.ndim
        return pl.BlockSpec(arr.shape, lambda j, n=n: (0,) * n)

    smem = pl.BlockSpec(memory_space=pltpu.SMEM)

    lw, x1 = pl.pallas_call(
        _layer1_kernel,
        grid=(L,),
        in_specs=[full(x2d), full(p1),
                  pl.BlockSpec((1, D_ATT, S), lambda j: (j, 0, 0)),
                  full(wsel), full(W_root1),
                  pl.BlockSpec((1, 1, L), lambda j: (j, 0, 0)),
                  full(qmask), full(spkf), smem, smem],
        out_specs=(pl.BlockSpec((1, LS, S), lambda j: (j, 0, 0)),
                   pl.BlockSpec((1, S, D_L), lambda j: (j, 0, 0))),
        out_shape=(jax.ShapeDtypeStruct((L, LS, S), f32),
                   jax.ShapeDtypeStruct((L, S, D_L), f32)),
    )(x2d, p1, p2t, wsel, W_root1, gwt3, qmask, spkf, spk, lng)

    x1_2d = x1.reshape(LS, D_L)
    x2 = pl.pallas_call(
        _layer2_kernel,
        grid=(L,),
        in_specs=[pl.BlockSpec((1, LS, S), lambda j: (j, 0, 0)),
                  full(x1_2d), full(wrel2), full(W_root2)],
        out_specs=pl.BlockSpec((1, S, D_L), lambda j: (j, 0, 0)),
        out_shape=jax.ShapeDtypeStruct((L, S, D_L), f32),
    )(lw, x1_2d, wrel2, W_root2)

    return x2


# speedup vs baseline: 8.0335x; 1.0815x over previous
import jax
import jax.numpy as jnp
from jax.experimental import pallas as pl
from jax.experimental.pallas import tpu as pltpu

L = 32
S = 64
D_L = 128
D_ATT = 128
LS = L * S


def _prelude_kernel(g_ref, wq_ref, wk_ref, vg_ref, x_ref, wk1_ref, wk2_ref,
                    gwt_ref, p1_ref, p2t_ref):
    g = g_ref[...]
    q = jnp.dot(g, wq_ref[...], preferred_element_type=jnp.float32)
    k = jnp.dot(g, wk_ref[...], preferred_element_type=jnp.float32)
    t = jnp.tanh(q[:, None, :] + k[None, :, :])
    s = jnp.sum(t * vg_ref[0][None, None, :], axis=-1)
    s = s - jnp.max(s, axis=-1, keepdims=True)
    e = jnp.exp(s)
    gw = e / jnp.sum(e, axis=-1, keepdims=True)
    gwt_ref[...] = gw.T
    x2d = x_ref[...]
    p1_ref[...] = jnp.tanh(
        jnp.dot(x2d, wk1_ref[...], preferred_element_type=jnp.float32))
    p2 = jnp.tanh(
        jnp.dot(x2d, wk2_ref[...], preferred_element_type=jnp.float32))
    p2t_ref[...] = jnp.transpose(p2.reshape(L, S, D_ATT), (0, 2, 1))


def _fused_kernel(x_ref, p1_ref, p2t_ref, wsel_ref, wroot1_ref, gwt_ref,
                  qmask_ref, tmask_ref, spkf_ref, wrel2_ref, wroot2_ref,
                  spk_ref, x2_ref, lw_ref, x1_ref):
    p = pl.program_id(0)
    j = pl.program_id(1)

    @pl.when(p == 0)
    def _layer1():
        sc = jnp.dot(p1_ref[...], p2t_ref[0],
                     preferred_element_type=jnp.float32) * (
                         1.0 / jnp.sqrt(jnp.float32(D_ATT)))
        e = jnp.exp(sc) * tmask_ref[0]
        ssum = jnp.dot(e, jnp.ones((S, 1), jnp.float32),
                       preferred_element_type=jnp.float32)
        lw = e * (qmask_ref[...] / ssum)
        lw_ref[j] = lw
        msg = jax.lax.dot_general(
            lw.reshape(L, S, S), x_ref[...].reshape(L, S, D_L),
            (((2,), (1,)), ((0,), (0,))),
            preferred_element_type=jnp.float32)
        spr = spkf_ref[...]
        ilt = (jax.lax.broadcasted_iota(jnp.int32, (1, L), 1) < j
               ).astype(jnp.float32)
        wg = jnp.concatenate(
            [(1.0 - spr) * ilt, (1.0 - spr) * (1.0 - ilt),
             spr * ilt, spr * (1.0 - ilt)], axis=0) * gwt_ref[0]
        acc4 = jax.lax.dot_general(
            wg, msg, (((1,), (0,)), ((), ())),
            preferred_element_type=jnp.float32)
        is0 = spk_ref[j] == 0

        def rel(a, d):
            return jnp.where(is0, wsel_ref[a, 0, d], wsel_ref[a, 1, d])

        agg = (jnp.dot(acc4[0], rel(0, 0),
                       preferred_element_type=jnp.float32)
               + jnp.dot(acc4[1], rel(0, 1),
                         preferred_element_type=jnp.float32)
               + jnp.dot(acc4[2], rel(1, 0),
                         preferred_element_type=jnp.float32)
               + jnp.dot(acc4[3], rel(1, 1),
                         preferred_element_type=jnp.float32))
        x1_ref[j] = jnp.dot(x_ref[pl.ds(j * S, S)], wroot1_ref[...],
                            preferred_element_type=jnp.float32) + agg

    @pl.when(p == 1)
    def _layer2():
        msg = jax.lax.dot_general(
            lw_ref[j].reshape(L, S, S), x1_ref[...],
            (((2,), (1,)), ((0,), (0,))),
            preferred_element_type=jnp.float32)
        msum = jnp.sum(msg, axis=0)
        x2_ref[0] = (jnp.dot(x1_ref[j], wroot2_ref[...],
                             preferred_element_type=jnp.float32)
                     + jnp.dot(msum, wrel2_ref[...],
                               preferred_element_type=jnp.float32))


@jax.jit
def kernel(global_features, local_features, speaker, length, Wq_g, Wk_g,
           v_g, Wk1_l, Wk2_l, W_rel1, W_root1, W_rel2, W_root2):
    f32 = jnp.float32
    x2d = local_features.astype(f32).reshape(LS, D_L)
    wsel = jnp.concatenate([W_rel1[0:4], W_rel1[64:68]], axis=0)
    wsel = wsel.reshape(2, 2, 2, D_L, D_L)
    vg2 = v_g.reshape(1, D_ATT)
    spk = speaker.astype(jnp.int32)
    lng = length.astype(jnp.int32)
    spkf = speaker.astype(f32).reshape(1, L)
    wrel2 = W_rel2.reshape(D_L, D_L)
    sr = jnp.arange(S, dtype=jnp.int32)
    qmask = (sr[None, :] < lng[:, None]).astype(f32).reshape(LS, 1)
    tmask = (sr[None, :] < lng[:, None]).astype(f32).reshape(L, 1, S)

    gwt, p1, p2t = pl.pallas_call(
        _prelude_kernel,
        out_shape=(jax.ShapeDtypeStruct((L, L), f32),
                   jax.ShapeDtypeStruct((LS, D_ATT), f32),
                   jax.ShapeDtypeStruct((L, D_ATT, S), f32)),
    )(global_features, Wq_g, Wk_g, vg2, x2d, Wk1_l, Wk2_l)
    gwt3 = gwt.reshape(L, 1, L)

    def full(arr):
        n = arr.ndim
        return pl.BlockSpec(arr.shape, lambda p, j, n=n: (0,) * n)

    x2 = pl.pallas_call(
        _fused_kernel,
        grid=(2, L),
        in_specs=[full(x2d), full(p1),
                  pl.BlockSpec((1, D_ATT, S), lambda p, j: (j, 0, 0)),
                  full(wsel), full(W_root1),
                  pl.BlockSpec((1, 1, L), lambda p, j: (j, 0, 0)),
                  full(qmask),
                  pl.BlockSpec((1, 1, S), lambda p, j: (j, 0, 0)),
                  full(spkf), full(wrel2), full(W_root2),
                  pl.BlockSpec(memory_space=pltpu.SMEM)],
        out_specs=pl.BlockSpec((1, S, D_L), lambda p, j: (j, 0, 0)),
        out_shape=jax.ShapeDtypeStruct((L, S, D_L), f32),
        scratch_shapes=[pltpu.VMEM((L, LS, S), f32),
                        pltpu.VMEM((L, S, D_L), f32)],
    )(x2d, p1, p2t, wsel, W_root1, gwt3, qmask, tmask, spkf, wrel2,
      W_root2, spk)

    return x2


# speedup vs baseline: 8.9761x; 1.1173x over previous
import jax
import jax.numpy as jnp
from jax.experimental import pallas as pl
from jax.experimental.pallas import tpu as pltpu

L = 32
S = 64
D_L = 128
D_ATT = 128
LS = L * S


def _fused_kernel(x_ref, g_ref, wq_ref, wk_ref, vg_ref, wk1_ref, wk2_ref,
                  wra_ref, wrb_ref, wroot1_ref, qmask_ref, tmask_ref,
                  spkf_ref, wrel2_ref, wroot2_ref, spk_ref,
                  x2_ref, lw_ref, x1_ref, p1_ref, p2t_ref, gwt_ref):
    p = pl.program_id(0)
    j = pl.program_id(1)

    @pl.when(jnp.logical_and(p == 0, j == 0))
    def _prelude():
        g = g_ref[...]
        q = jnp.dot(g, wq_ref[...], preferred_element_type=jnp.float32)
        k = jnp.dot(g, wk_ref[...], preferred_element_type=jnp.float32)
        t = jnp.tanh(q[:, None, :] + k[None, :, :])
        s = jnp.sum(t * vg_ref[0][None, None, :], axis=-1)
        s = s - jnp.max(s, axis=-1, keepdims=True)
        e = jnp.exp(s)
        gw = e / jnp.sum(e, axis=-1, keepdims=True)
        gwt_ref[...] = gw.T
        x2d = x_ref[...]
        p1_ref[...] = jnp.tanh(
            jnp.dot(x2d, wk1_ref[...], preferred_element_type=jnp.float32))
        p2 = jnp.tanh(
            jnp.dot(x2d, wk2_ref[...], preferred_element_type=jnp.float32))
        p2t_ref[...] = jnp.transpose(p2.reshape(L, S, D_ATT), (0, 2, 1))

    @pl.when(p == 0)
    def _layer1():
        sc = jnp.dot(p1_ref[...], p2t_ref[j],
                     preferred_element_type=jnp.float32) * (
                         1.0 / jnp.sqrt(jnp.float32(D_ATT)))
        e = jnp.exp(sc) * tmask_ref[0]
        ssum = jnp.dot(e, jnp.ones((S, 1), jnp.float32),
                       preferred_element_type=jnp.float32)
        lw = e * (qmask_ref[...] / ssum)
        lw_ref[j] = lw
        msg = jax.lax.dot_general(
            lw.reshape(L, S, S), x_ref[...].reshape(L, S, D_L),
            (((2,), (1,)), ((0,), (0,))),
            preferred_element_type=jnp.float32)
        spr = spkf_ref[...]
        ilt = (jax.lax.broadcasted_iota(jnp.int32, (1, L), 1) < j
               ).astype(jnp.float32)
        wg = jnp.concatenate(
            [(1.0 - spr) * ilt, (1.0 - spr) * (1.0 - ilt),
             spr * ilt, spr * (1.0 - ilt)],
            axis=0) * gwt_ref[pl.ds(j, 1)]
        acc4 = jax.lax.dot_general(
            wg, msg, (((1,), (0,)), ((), ())),
            preferred_element_type=jnp.float32)
        is0 = spk_ref[j] == 0

        def rel(a, d):
            wr = wra_ref if a == 0 else wrb_ref
            return jnp.where(is0, wr[d], wr[2 + d])

        agg = (jnp.dot(acc4[0], rel(0, 0),
                       preferred_element_type=jnp.float32)
               + jnp.dot(acc4[1], rel(0, 1),
                         preferred_element_type=jnp.float32)
               + jnp.dot(acc4[2], rel(1, 0),
                         preferred_element_type=jnp.float32)
               + jnp.dot(acc4[3], rel(1, 1),
                         preferred_element_type=jnp.float32))
        x1_ref[j] = jnp.dot(x_ref[pl.ds(j * S, S)], wroot1_ref[...],
                            preferred_element_type=jnp.float32) + agg

    @pl.when(p == 1)
    def _layer2():
        msg = jax.lax.dot_general(
            lw_ref[j].reshape(L, S, S), x1_ref[...],
            (((2,), (1,)), ((0,), (0,))),
            preferred_element_type=jnp.float32)
        msum = jnp.sum(msg, axis=0)
        x2_ref[0] = (jnp.dot(x1_ref[j], wroot2_ref[...],
                             preferred_element_type=jnp.float32)
                     + jnp.dot(msum, wrel2_ref[...],
                               preferred_element_type=jnp.float32))


@jax.jit
def kernel(global_features, local_features, speaker, length, Wq_g, Wk_g,
           v_g, Wk1_l, Wk2_l, W_rel1, W_root1, W_rel2, W_root2):
    f32 = jnp.float32
    x2d = local_features.astype(f32).reshape(LS, D_L)
    vg2 = v_g.reshape(1, D_ATT)
    spk = speaker.astype(jnp.int32)
    lng = length.astype(jnp.int32)
    spkf = speaker.astype(f32).reshape(1, L)
    wrel2 = W_rel2.reshape(D_L, D_L)
    sr = jnp.arange(S, dtype=jnp.int32)
    qmask = (sr[None, :] < lng[:, None]).astype(f32).reshape(LS, 1)
    tmask = (sr[None, :] < lng[:, None]).astype(f32).reshape(L, 1, S)

    def full(arr):
        n = arr.ndim
        return pl.BlockSpec(arr.shape, lambda p, j, n=n: (0,) * n)

    x2 = pl.pallas_call(
        _fused_kernel,
        grid=(2, L),
        in_specs=[full(x2d), full(global_features), full(Wq_g), full(Wk_g),
                  full(vg2), full(Wk1_l), full(Wk2_l),
                  pl.BlockSpec((4, D_L, D_L), lambda p, j: (0, 0, 0)),
                  pl.BlockSpec((4, D_L, D_L), lambda p, j: (16, 0, 0)),
                  full(W_root1), full(qmask),
                  pl.BlockSpec((1, 1, S), lambda p, j: (j, 0, 0)),
                  full(spkf), full(wrel2), full(W_root2),
                  pl.BlockSpec(memory_space=pltpu.SMEM)],
        out_specs=pl.BlockSpec((1, S, D_L), lambda p, j: (j, 0, 0)),
        out_shape=jax.ShapeDtypeStruct((L, S, D_L), f32),
        scratch_shapes=[pltpu.VMEM((L, LS, S), f32),
                        pltpu.VMEM((L, S, D_L), f32),
                        pltpu.VMEM((LS, D_ATT), f32),
                        pltpu.VMEM((L, D_ATT, S), f32),
                        pltpu.VMEM((L, L), f32)],
    )(x2d, global_features, Wq_g, Wk_g, vg2, Wk1_l, Wk2_l,
      W_rel1, W_rel1, W_root1, qmask, tmask, spkf, wrel2, W_root2, spk)

    return x2
